# baseline (speedup 1.0000x reference)
import functools

import jax
import jax.numpy as jnp
from jax import lax
from jax.experimental import pallas as pl
from jax.experimental.pallas import tpu as pltpu
from jax.experimental.pallas import tpu_sc as plsc

D = 512
V = 128
G = 16
IMB = 1696


@functools.cache
def _make_kernel(N: int):
    info = plsc.get_sparse_core_info()
    NC, NS = info.num_cores, info.num_subcores
    NW = NC * NS
    mesh = plsc.VectorSubcoreMesh(core_axis_name="c", subcore_axis_name="s")

    @functools.partial(
        pl.kernel,
        mesh=mesh,
        compiler_params=pltpu.CompilerParams(needs_layout_passes=False),
        out_type=jax.ShapeDtypeStruct((N, 3 * D), jnp.float32),
        scratch_types=[
            pltpu.VMEM_SHARED((3 * V, D), jnp.float32),
            pltpu.VMEM((V, D), jnp.float32),
            pltpu.VMEM((IMB,), jnp.int32),
            pltpu.SemaphoreType.DMA,
        ],
    )
    def k(pitch_h, program_h, velocity_h, ptab_h, gtab_h, vtab_h, out_h,
          tabs_s, tab_v, idx_v, wsem):
        sid = lax.axis_index("s")
        wid = sid * NC + lax.axis_index("c")

        t = (wid * 3) // NW
        start_w = (t * NW + 2) // 3
        r = wid - start_w
        per_w = jnp.where(t == 2, 1632, 1472)
        last_r = jnp.where(t == 2, 9, 10)
        tok_start = r * per_w
        cnt = jnp.where(r == last_r, N - last_r * per_w, per_w)

        @pl.when(sid == 0)
        def _():
            pltpu.sync_copy(ptab_h, tabs_s.at[pl.ds(0, V)])
            pltpu.sync_copy(gtab_h, tabs_s.at[pl.ds(V, V)])
            pltpu.sync_copy(vtab_h, tabs_s.at[pl.ds(2 * V, V)])

        idx_h = (pitch_h, program_h, velocity_h)
        clamp = jnp.minimum(tok_start, N - IMB)
        shift = tok_start - clamp
        for tt in range(3):
            @pl.when(t == tt)
            def _(tt=tt):
                pltpu.sync_copy(idx_h[tt].at[pl.ds(clamp, IMB)], idx_v)

        plsc.subcore_barrier()

        pltpu.sync_copy(tabs_s.at[pl.ds(t * V, V)], tab_v)

        col = t * D

        def body(gi, carry):
            off = gi * G
            idx16 = idx_v[pl.ds(shift + off, G)]
            for l in range(G):
                row = idx16[l]
                pltpu.async_copy(
                    tab_v.at[pl.ds(row, 1), :],
                    out_h.at[pl.ds(tok_start + off + l, 1), pl.ds(col, D)],
                    wsem,
                )
            return carry

        lax.fori_loop(0, cnt // G, body, 0)

        def drain(gi, carry):
            pltpu.make_async_copy(
                tab_v.at[pl.ds(0, G), :],
                out_h.at[pl.ds(0, G), pl.ds(0, D)],
                wsem,
            ).wait()
            return carry

        lax.fori_loop(0, cnt // G, drain, 0)

    return k


def kernel(pitch, program, velocity, pitch_table, program_table, velocity_table):
    B, S = pitch.shape
    N = B * S
    p = pitch.reshape(N).astype(jnp.int32)
    g = program.reshape(N).astype(jnp.int32)
    v = velocity.reshape(N).astype(jnp.int32)
    out = _make_kernel(N)(p, g, v, pitch_table, program_table, velocity_table)
    return out.reshape(B, S, 3 * D)


# speedup vs baseline: 1.0031x; 1.0031x over previous
import functools

import jax
import jax.numpy as jnp
from jax import lax
from jax.experimental import pallas as pl
from jax.experimental.pallas import tpu as pltpu
from jax.experimental.pallas import tpu_sc as plsc

D = 512
V = 128
G = 16


@functools.cache
def _make_kernel(N: int):
    info = plsc.get_sparse_core_info()
    NC, NS = info.num_cores, info.num_subcores
    NW = NC * NS
    nw01, nw2 = (NW + 2) // 3, NW - 2 * ((NW + 2) // 3)
    pw01 = N // nw01 // 32 * 32
    pw2 = N // nw2 // 32 * 32
    last01 = N - (nw01 - 1) * pw01
    last2 = N - (nw2 - 1) * pw2
    IMB = max(pw01, pw2, last01, last2)
    mesh = plsc.VectorSubcoreMesh(core_axis_name="c", subcore_axis_name="s")

    @functools.partial(
        pl.kernel,
        mesh=mesh,
        compiler_params=pltpu.CompilerParams(needs_layout_passes=False),
        out_type=jax.ShapeDtypeStruct((N, 3 * D), jnp.float32),
        scratch_types=[
            pltpu.VMEM_SHARED((3 * V, D), jnp.float32),
            pltpu.VMEM((V, D), jnp.float32),
            pltpu.VMEM((IMB,), jnp.int32),
            pltpu.SemaphoreType.DMA,
        ],
    )
    def k(pitch_h, program_h, velocity_h, ptab_h, gtab_h, vtab_h, out_h,
          tabs_s, tab_v, idx_v, wsem):
        sid = lax.axis_index("s")
        wid = sid * NC + lax.axis_index("c")

        t = (wid * 3) // NW
        start_w = (t * NW + 2) // 3
        r = wid - start_w
        per_w = jnp.where(t == 2, pw2, pw01)
        last_r = jnp.where(t == 2, nw2 - 1, nw01 - 1)
        tok_start = r * per_w
        cnt = jnp.where(r == last_r, N - last_r * per_w, per_w)

        @pl.when(sid == 0)
        def _():
            pltpu.sync_copy(ptab_h, tabs_s.at[pl.ds(0, V)])
            pltpu.sync_copy(gtab_h, tabs_s.at[pl.ds(V, V)])
            pltpu.sync_copy(vtab_h, tabs_s.at[pl.ds(2 * V, V)])

        idx_h = (pitch_h, program_h, velocity_h)
        clamp = jnp.minimum(tok_start, N - IMB)
        shift = tok_start - clamp
        for tt in range(3):
            @pl.when(t == tt)
            def _(tt=tt):
                pltpu.sync_copy(idx_h[tt].at[pl.ds(clamp, IMB)], idx_v)

        plsc.subcore_barrier()

        pltpu.sync_copy(tabs_s.at[pl.ds(t * V, V)], tab_v)

        col = t * D

        def body(gi, carry):
            off = gi * G
            idx16 = idx_v[pl.ds(shift + off, G)]
            for l in range(G):
                row = idx16[l]
                pltpu.async_copy(
                    tab_v.at[pl.ds(row, 1), :],
                    out_h.at[pl.ds(tok_start + off + l, 1), pl.ds(col, D)],
                    wsem,
                )
            return carry

        lax.fori_loop(0, cnt // G, body, 0)

        def drain(gi, carry):
            pltpu.make_async_copy(
                tab_v.at[pl.ds(0, G), :],
                out_h.at[pl.ds(0, G), pl.ds(0, D)],
                wsem,
            ).wait()
            return carry

        lax.fori_loop(0, cnt // G, drain, 0)

    return k


def kernel(pitch, program, velocity, pitch_table, program_table, velocity_table):
    B, S = pitch.shape
    N = B * S
    p = pitch.reshape(N).astype(jnp.int32)
    g = program.reshape(N).astype(jnp.int32)
    v = velocity.reshape(N).astype(jnp.int32)
    out = _make_kernel(N)(p, g, v, pitch_table, program_table, velocity_table)
    return out.reshape(B, S, 3 * D)
